# Initial kernel scaffold; baseline (speedup 1.0000x reference)
import jax
import jax.numpy as jnp
from jax.experimental import pallas as pl


def kernel(x_p2, x_p3, x_p4, x_p5, boxes):
    raise NotImplementedError("write your pallas kernel here")


# speedup vs baseline: 35.6360x; 35.6360x over previous
import functools

import jax
import jax.numpy as jnp
from jax import lax
from jax.experimental import pallas as pl
from jax.experimental.pallas import tpu as pltpu
from jax.experimental.pallas import tpu_sc as plsc

OUT = 7
S = 14
NBOX = 512
C = 256
BASE0 = 0
BASE1 = 2 * 256 * 256
BASE2 = BASE1 + 2 * 128 * 128
BASE3 = BASE2 + 2 * 64 * 64
NROWS = BASE3 + 2 * 32 * 32


def _prep_body(bx_ref, idx_ref, w_ref):
    f32 = jnp.float32
    x1 = bx_ref[0:1, :]
    y1 = bx_ref[1:2, :]
    x2 = bx_ref[2:3, :]
    y2 = bx_ref[3:4, :]
    area = (x2 - x1) * (y2 - y1)
    v = jnp.sqrt(area) / 224.0 + 1e-8
    lvl = ((v >= 0.5).astype(jnp.int32) + (v >= 1.0).astype(jnp.int32)
           + (v >= 2.0).astype(jnp.int32))
    scale = jnp.where(lvl == 0, f32(0.25),
             jnp.where(lvl == 1, f32(0.125),
              jnp.where(lvl == 2, f32(0.0625), f32(0.03125))))
    Hn = jnp.where(lvl == 0, 256, jnp.where(lvl == 1, 128,
          jnp.where(lvl == 2, 64, 32)))
    base = jnp.where(lvl == 0, BASE0, jnp.where(lvl == 1, BASE1,
            jnp.where(lvl == 2, BASE2, BASE3)))
    Hf = Hn.astype(f32)
    x1s = x1 * scale - 0.5
    y1s = y1 * scale - 0.5
    bw = (x2 * scale - 0.5 - x1s) / f32(OUT)
    bh = (y2 * scale - 0.5 - y1s) / f32(OUT)

    s = lax.broadcasted_iota(jnp.int32, (S * S, NBOX), 0)
    box = lax.broadcasted_iota(jnp.int32, (S * S, NBOX), 1)
    sy = s // S
    sx = s - sy * S
    yy = y1s + (sy.astype(f32) * 0.5 + 0.25) * bh
    xx = x1s + (sx.astype(f32) * 0.5 + 0.25) * bw
    valid = ((yy >= -1.0) & (yy <= Hf)) & ((xx >= -1.0) & (xx <= Hf))
    y = jnp.maximum(yy, 0.0)
    x = jnp.maximum(xx, 0.0)
    y0 = jnp.floor(y).astype(jnp.int32)
    x0 = jnp.floor(x).astype(jnp.int32)
    ych = y0 >= Hn - 1
    xch = x0 >= Hn - 1
    y_low = jnp.where(ych, Hn - 1, y0)
    y_high = jnp.where(ych, Hn - 1, y0 + 1)
    yc = jnp.where(ych, Hf - 1.0, y)
    x_low = jnp.where(xch, Hn - 1, x0)
    x_high = jnp.where(xch, Hn - 1, x0 + 1)
    xc = jnp.where(xch, Hf - 1.0, x)
    ly = yc - y_low.astype(f32)
    lx = xc - x_low.astype(f32)
    hy = 1.0 - ly
    hx = 1.0 - lx
    vm = jnp.where(valid, f32(0.25), f32(0.0))

    b = box // 256
    rowbase = base + b * (Hn * Hn)
    idx_ref[0] = rowbase + y_low * Hn + x_low
    idx_ref[1] = rowbase + y_low * Hn + x_high
    idx_ref[2] = rowbase + y_high * Hn + x_low
    idx_ref[3] = rowbase + y_high * Hn + x_high
    w_ref[0] = hy * hx * vm
    w_ref[1] = hy * lx * vm
    w_ref[2] = ly * hx * vm
    w_ref[3] = ly * lx * vm


_prep = pl.pallas_call(
    _prep_body,
    out_shape=[
        jax.ShapeDtypeStruct((4, S * S, NBOX), jnp.int32),
        jax.ShapeDtypeStruct((4, S * S, NBOX), jnp.float32),
    ],
)


def _lane_bcast(vec, j):
    return lax.gather(
        vec,
        jnp.full((16, 1), j, jnp.int32),
        lax.GatherDimensionNumbers(
            offset_dims=(), collapsed_slice_dims=(0,), start_index_map=(0,)),
        (1,),
        mode=lax.GatherScatterMode.PROMISE_IN_BOUNDS,
    )


def _sc_pool(table, idx3, wflat):
    mesh = plsc.VectorSubcoreMesh(core_axis_name="c", subcore_axis_name="s")

    @functools.partial(
        pl.kernel,
        mesh=mesh,
        out_type=jax.ShapeDtypeStruct((NBOX, 49, C), jnp.float32),
        scratch_types=[
            pltpu.VMEM((7, 112), jnp.int32),
            pltpu.VMEM((784,), jnp.float32),
            pltpu.VMEM((112, C), jnp.float32),
            pltpu.VMEM((49, C), jnp.float32),
            pltpu.SemaphoreType.DMA,
        ],
    )
    def k(table_hbm, idx_hbm, w_hbm, out_hbm, idx_v, w_v, rows_v, acc_v, sem):
        wid = lax.axis_index("s") * 2 + lax.axis_index("c")

        def box_body(t, carry):
            bi = wid * 16 + t
            pltpu.sync_copy(idx_hbm.at[bi], idx_v)
            pltpu.sync_copy(w_hbm.at[bi], w_v)

            def chunk_body(c, carry2):
                pltpu.async_copy(table_hbm.at[idx_v.at[c]], rows_v, sem).wait()

                def px_body(p, carry3):
                    off = c * 112 + p * 16
                    wvec = w_v[pl.ds(off, 16)]
                    acc = [jnp.zeros((16,), jnp.float32)
                           for _ in range(C // 16)]
                    for j in range(16):
                        wj = _lane_bcast(wvec, j)
                        r = p * 16 + j
                        for cc in range(C // 16):
                            acc[cc] = acc[cc] + wj * rows_v[r, pl.ds(cc * 16, 16)]
                    pg = c * 7 + p
                    for cc in range(C // 16):
                        acc_v[pg, pl.ds(cc * 16, 16)] = acc[cc]
                    return carry3

                return lax.fori_loop(0, 7, px_body, carry2)

            lax.fori_loop(0, 7, chunk_body, 0)
            pltpu.sync_copy(acc_v, out_hbm.at[bi])
            return carry

        lax.fori_loop(0, 16, box_body, 0)

    return k(table, idx3, wflat)


def kernel(x_p2, x_p3, x_p4, x_p5, boxes):
    table = jnp.concatenate([
        x_p2.transpose(0, 2, 3, 1).reshape(-1, C),
        x_p3.transpose(0, 2, 3, 1).reshape(-1, C),
        x_p4.transpose(0, 2, 3, 1).reshape(-1, C),
        x_p5.transpose(0, 2, 3, 1).reshape(-1, C),
    ], axis=0)

    bxs = boxes.reshape(NBOX, 4).T
    idx4, w4 = _prep(bxs)

    def reorder(a):
        a = a.transpose(2, 1, 0).reshape(NBOX, OUT, 2, OUT, 2, 4)
        return a.transpose(0, 1, 3, 2, 4, 5).reshape(NBOX, 49 * 16)

    idx = reorder(idx4).reshape(NBOX, 7, 112)
    w = reorder(w4)

    out = _sc_pool(table, idx, w)
    return out.reshape(NBOX, OUT, OUT, C).transpose(0, 3, 1, 2)


# speedup vs baseline: 43.3998x; 1.2179x over previous
import functools

import jax
import jax.numpy as jnp
from jax import lax
from jax.experimental import pallas as pl
from jax.experimental.pallas import tpu as pltpu
from jax.experimental.pallas import tpu_sc as plsc

OUT = 7
S = 14
NBOX = 512
C = 256
BASE0 = 0
BASE1 = 2 * 256 * 256
BASE2 = BASE1 + 2 * 128 * 128
BASE3 = BASE2 + 2 * 64 * 64
NROWS = BASE3 + 2 * 32 * 32


def _prep_body(bx_ref, idx_ref, w_ref):
    f32 = jnp.float32
    x1 = bx_ref[0:1, :]
    y1 = bx_ref[1:2, :]
    x2 = bx_ref[2:3, :]
    y2 = bx_ref[3:4, :]
    area = (x2 - x1) * (y2 - y1)
    v = jnp.sqrt(area) / 224.0 + 1e-8
    lvl = ((v >= 0.5).astype(jnp.int32) + (v >= 1.0).astype(jnp.int32)
           + (v >= 2.0).astype(jnp.int32))
    scale = jnp.where(lvl == 0, f32(0.25),
             jnp.where(lvl == 1, f32(0.125),
              jnp.where(lvl == 2, f32(0.0625), f32(0.03125))))
    Hn = jnp.where(lvl == 0, 256, jnp.where(lvl == 1, 128,
          jnp.where(lvl == 2, 64, 32)))
    base = jnp.where(lvl == 0, BASE0, jnp.where(lvl == 1, BASE1,
            jnp.where(lvl == 2, BASE2, BASE3)))
    Hf = Hn.astype(f32)
    x1s = x1 * scale - 0.5
    y1s = y1 * scale - 0.5
    bw = (x2 * scale - 0.5 - x1s) / f32(OUT)
    bh = (y2 * scale - 0.5 - y1s) / f32(OUT)

    s = lax.broadcasted_iota(jnp.int32, (S * S, NBOX), 0)
    box = lax.broadcasted_iota(jnp.int32, (S * S, NBOX), 1)
    sy = s // S
    sx = s - sy * S
    yy = y1s + (sy.astype(f32) * 0.5 + 0.25) * bh
    xx = x1s + (sx.astype(f32) * 0.5 + 0.25) * bw
    valid = ((yy >= -1.0) & (yy <= Hf)) & ((xx >= -1.0) & (xx <= Hf))
    y = jnp.maximum(yy, 0.0)
    x = jnp.maximum(xx, 0.0)
    y0 = jnp.floor(y).astype(jnp.int32)
    x0 = jnp.floor(x).astype(jnp.int32)
    ych = y0 >= Hn - 1
    xch = x0 >= Hn - 1
    y_low = jnp.where(ych, Hn - 1, y0)
    y_high = jnp.where(ych, Hn - 1, y0 + 1)
    yc = jnp.where(ych, Hf - 1.0, y)
    x_low = jnp.where(xch, Hn - 1, x0)
    x_high = jnp.where(xch, Hn - 1, x0 + 1)
    xc = jnp.where(xch, Hf - 1.0, x)
    ly = yc - y_low.astype(f32)
    lx = xc - x_low.astype(f32)
    hy = 1.0 - ly
    hx = 1.0 - lx
    vm = jnp.where(valid, f32(0.25), f32(0.0))

    b = box // 256
    rowbase = base + b * (Hn * Hn)
    idx_ref[0] = rowbase + y_low * Hn + x_low
    idx_ref[1] = rowbase + y_low * Hn + x_high
    idx_ref[2] = rowbase + y_high * Hn + x_low
    idx_ref[3] = rowbase + y_high * Hn + x_high
    w_ref[0] = hy * hx * vm
    w_ref[1] = hy * lx * vm
    w_ref[2] = ly * hx * vm
    w_ref[3] = ly * lx * vm


_prep = pl.pallas_call(
    _prep_body,
    out_shape=[
        jax.ShapeDtypeStruct((4, S * S, NBOX), jnp.int32),
        jax.ShapeDtypeStruct((4, S * S, NBOX), jnp.float32),
    ],
)


def _lane_bcast(vec, j):
    return lax.gather(
        vec,
        jnp.full((16, 1), j, jnp.int32),
        lax.GatherDimensionNumbers(
            offset_dims=(), collapsed_slice_dims=(0,), start_index_map=(0,)),
        (1,),
        mode=lax.GatherScatterMode.PROMISE_IN_BOUNDS,
    )


def _sc_pool(table, idx3, wflat):
    mesh = plsc.VectorSubcoreMesh(core_axis_name="c", subcore_axis_name="s")

    @functools.partial(
        pl.kernel,
        mesh=mesh,
        out_type=jax.ShapeDtypeStruct((NBOX, 49, C), jnp.float32),
        scratch_types=[
            pltpu.VMEM((7, 112), jnp.int32),
            pltpu.VMEM((784,), jnp.float32),
            pltpu.VMEM((2, 112, C), jnp.float32),
            pltpu.VMEM((49, C), jnp.float32),
            pltpu.SemaphoreType.DMA,
            pltpu.SemaphoreType.DMA,
        ],
    )
    def k(table_hbm, idx_hbm, w_hbm, out_hbm, idx_v, w_v, rows_v, acc_v,
          sem0, sem1):
        wid = lax.axis_index("s") * 2 + lax.axis_index("c")
        sems = [sem0, sem1]

        def box_body(t, carry):
            bi = wid * 16 + t
            pltpu.sync_copy(idx_hbm.at[bi], idx_v)
            pltpu.sync_copy(w_hbm.at[bi], w_v)

            cps = [None, None]
            cps[0] = pltpu.async_copy(
                table_hbm.at[idx_v.at[0]], rows_v.at[0], sems[0])
            for c in range(7):
                b = c % 2
                if c + 1 < 7:
                    nb = (c + 1) % 2
                    cps[nb] = pltpu.async_copy(
                        table_hbm.at[idx_v.at[c + 1]], rows_v.at[nb], sems[nb])
                cps[b].wait()

                def px_body(p, carry3, c=c, b=b):
                    off = c * 112 + p * 16
                    wvec = w_v[pl.ds(off, 16)]
                    acc = [jnp.zeros((16,), jnp.float32)
                           for _ in range(C // 16)]
                    for j in range(16):
                        wj = _lane_bcast(wvec, j)
                        r = p * 16 + j
                        for cc in range(C // 16):
                            acc[cc] = acc[cc] + wj * rows_v[b, r, pl.ds(cc * 16, 16)]
                    pg = c * 7 + p
                    for cc in range(C // 16):
                        acc_v[pg, pl.ds(cc * 16, 16)] = acc[cc]
                    return carry3

                lax.fori_loop(0, 7, px_body, 0)
            pltpu.sync_copy(acc_v, out_hbm.at[bi])
            return carry

        lax.fori_loop(0, 16, box_body, 0)

    return k(table, idx3, wflat)


def kernel(x_p2, x_p3, x_p4, x_p5, boxes):
    table = jnp.concatenate([
        x_p2.transpose(0, 2, 3, 1).reshape(-1, C),
        x_p3.transpose(0, 2, 3, 1).reshape(-1, C),
        x_p4.transpose(0, 2, 3, 1).reshape(-1, C),
        x_p5.transpose(0, 2, 3, 1).reshape(-1, C),
    ], axis=0)

    bxs = boxes.reshape(NBOX, 4).T
    idx4, w4 = _prep(bxs)

    def reorder(a):
        a = a.transpose(2, 1, 0).reshape(NBOX, OUT, 2, OUT, 2, 4)
        return a.transpose(0, 1, 3, 2, 4, 5).reshape(NBOX, 49 * 16)

    idx = reorder(idx4).reshape(NBOX, 7, 112)
    w = reorder(w4)

    out = _sc_pool(table, idx, w)
    return out.reshape(NBOX, OUT, OUT, C).transpose(0, 3, 1, 2)
